# Initial kernel scaffold; baseline (speedup 1.0000x reference)
import jax
import jax.numpy as jnp
from jax.experimental import pallas as pl


def kernel(x, edge_index, W_pre, b_pre, W1, b1, W2, b2):
    raise NotImplementedError("write your pallas kernel here")


# speedup vs baseline: 4.5422x; 4.5422x over previous
import functools

import jax
import jax.numpy as jnp
from jax import lax
from jax.experimental import pallas as pl
from jax.experimental.pallas import tpu as pltpu
from jax.experimental.pallas import tpu_sc as plsc

N = 10000
D = 128
E = 320000

NC = 2
NS = 16
NW = NC * NS

E_TILE = E // NW
CHUNK = 80
N_CHUNKS = E_TILE // CHUNK
ROWS_PER_TILE = 624
TAIL_ROW = NS * ROWS_PER_TILE
TAIL = N - TAIL_ROW
ZROWS = 48

BLOCK_M = 1000


def _mm_body(x_ref, w_ref, b_ref, o_ref, *, relu):
    acc = jnp.dot(x_ref[...], w_ref[...], preferred_element_type=jnp.float32)
    acc = acc + b_ref[...]
    if relu:
        acc = jnp.maximum(acc, 0.0)
    o_ref[...] = acc


def _mm(x, wt, b, relu=False):
    grid = (N // BLOCK_M,)
    return pl.pallas_call(
        functools.partial(_mm_body, relu=relu),
        grid=grid,
        in_specs=[
            pl.BlockSpec((BLOCK_M, D), lambda i: (i, 0)),
            pl.BlockSpec((D, D), lambda i: (0, 0)),
            pl.BlockSpec((1, D), lambda i: (0, 0)),
        ],
        out_specs=pl.BlockSpec((BLOCK_M, D), lambda i: (i, 0)),
        out_shape=jax.ShapeDtypeStruct((N, D), jnp.float32),
    )(x, wt, b.reshape(1, D))


def _agg_mm_body(h_ref, p0_ref, p1_ref, w_ref, b_ref, o_ref, *, relu):
    s = h_ref[...] + p0_ref[...] + p1_ref[...]
    acc = jnp.dot(s, w_ref[...], preferred_element_type=jnp.float32)
    acc = acc + b_ref[...]
    if relu:
        acc = jnp.maximum(acc, 0.0)
    o_ref[...] = acc


def _agg_mm(h, parts, wt, b, relu=False):
    grid = (N // BLOCK_M,)
    return pl.pallas_call(
        functools.partial(_agg_mm_body, relu=relu),
        grid=grid,
        in_specs=[
            pl.BlockSpec((BLOCK_M, D), lambda i: (i, 0)),
            pl.BlockSpec((BLOCK_M, D), lambda i: (i, 0)),
            pl.BlockSpec((BLOCK_M, D), lambda i: (i, 0)),
            pl.BlockSpec((D, D), lambda i: (0, 0)),
            pl.BlockSpec((1, D), lambda i: (0, 0)),
        ],
        out_specs=pl.BlockSpec((BLOCK_M, D), lambda i: (i, 0)),
        out_shape=jax.ShapeDtypeStruct((N, D), jnp.float32),
    )(h, parts[0], parts[1], wt, b.reshape(1, D))


def _sc_agg(h, src, dst):
    mesh = plsc.VectorSubcoreMesh(
        core_axis_name="c", subcore_axis_name="s", num_cores=NC, num_subcores=NS
    )

    @functools.partial(
        pl.kernel,
        out_type=jax.ShapeDtypeStruct((NC, N, D), jnp.float32),
        mesh=mesh,
        scratch_types=[
            pltpu.VMEM((CHUNK,), jnp.int32),
            pltpu.VMEM((CHUNK,), jnp.int32),
            pltpu.VMEM((CHUNK, D), jnp.float32),
            pltpu.VMEM((ZROWS, D), jnp.float32),
            pltpu.VMEM_SHARED((N, D), jnp.float32),
            pltpu.SemaphoreType.DMA,
        ],
    )
    def k(h_hbm, src_hbm, dst_hbm, out_hbm, src_v, dst_v, rows_v, zeros_v,
          acc_sh, sem):
        cid = lax.axis_index("c")
        sid = lax.axis_index("s")

        @pl.loop(0, ZROWS)
        def _(r):
            @pl.loop(0, D, step=16)
            def _(j):
                zeros_v[r, pl.ds(j, 16)] = jnp.zeros((16,), jnp.float32)

        row0 = sid * ROWS_PER_TILE

        @pl.loop(0, ROWS_PER_TILE, step=ZROWS)
        def _(r0):
            pltpu.sync_copy(zeros_v, acc_sh.at[pl.ds(row0 + r0, ZROWS)])

        @pl.when(sid == NS - 1)
        def _():
            pltpu.sync_copy(
                zeros_v.at[pl.ds(0, TAIL)], acc_sh.at[pl.ds(TAIL_ROW, TAIL)]
            )

        plsc.subcore_barrier()

        base = (cid * NS + sid) * E_TILE

        @pl.loop(0, N_CHUNKS)
        def _(ci):
            off = base + ci * CHUNK
            pltpu.sync_copy(src_hbm.at[pl.ds(off, CHUNK)], src_v)
            pltpu.sync_copy(dst_hbm.at[pl.ds(off, CHUNK)], dst_v)
            pltpu.async_copy(h_hbm.at[src_v], rows_v, sem).wait()
            pltpu.sync_copy(rows_v, acc_sh.at[dst_v], add=True)

        plsc.subcore_barrier()

        pltpu.sync_copy(
            acc_sh.at[pl.ds(row0, ROWS_PER_TILE)],
            out_hbm.at[cid, pl.ds(row0, ROWS_PER_TILE)],
        )

        @pl.when(sid == NS - 1)
        def _():
            pltpu.sync_copy(
                acc_sh.at[pl.ds(TAIL_ROW, TAIL)],
                out_hbm.at[cid, pl.ds(TAIL_ROW, TAIL)],
            )

    return k(h, src, dst)


@jax.jit
def kernel(x, edge_index, W_pre, b_pre, W1, b1, W2, b2):
    src = edge_index[0]
    dst = edge_index[1]
    h0 = _mm(x, W_pre.T, b_pre)
    p = _sc_agg(h0, src, dst)
    h1 = _agg_mm(h0, (p[0], p[1]), W1.T, b1, relu=True)
    q = _sc_agg(h1, src, dst)
    out = _agg_mm(h1, (q[0], q[1]), W2.T, b2)
    return out


# speedup vs baseline: 8.0241x; 1.7666x over previous
import functools

import jax
import jax.numpy as jnp
from jax import lax
from jax.experimental import pallas as pl
from jax.experimental.pallas import tpu as pltpu
from jax.experimental.pallas import tpu_sc as plsc

N = 10000
D = 128
E = 320000

NC = 2
NS = 16
NW = NC * NS

E_TILE = E // NW
CHUNK = 80
N_CHUNKS = E_TILE // CHUNK
ROWS_PER_TILE = 624
TAIL_ROW = NS * ROWS_PER_TILE
TAIL = N - TAIL_ROW
ZROWS = 16

BLOCK_M = 1000


def _mm_body(x_ref, w_ref, b_ref, o_ref, *, relu):
    acc = jnp.dot(x_ref[...], w_ref[...], preferred_element_type=jnp.float32)
    acc = acc + b_ref[...]
    if relu:
        acc = jnp.maximum(acc, 0.0)
    o_ref[...] = acc


def _mm(x, wt, b, relu=False):
    grid = (N // BLOCK_M,)
    return pl.pallas_call(
        functools.partial(_mm_body, relu=relu),
        grid=grid,
        in_specs=[
            pl.BlockSpec((BLOCK_M, D), lambda i: (i, 0)),
            pl.BlockSpec((D, D), lambda i: (0, 0)),
            pl.BlockSpec((1, D), lambda i: (0, 0)),
        ],
        out_specs=pl.BlockSpec((BLOCK_M, D), lambda i: (i, 0)),
        out_shape=jax.ShapeDtypeStruct((N, D), jnp.float32),
    )(x, wt, b.reshape(1, D))


def _agg_mm_body(h_ref, p0_ref, p1_ref, w_ref, b_ref, o_ref, *, relu):
    s = h_ref[...] + p0_ref[...] + p1_ref[...]
    acc = jnp.dot(s, w_ref[...], preferred_element_type=jnp.float32)
    acc = acc + b_ref[...]
    if relu:
        acc = jnp.maximum(acc, 0.0)
    o_ref[...] = acc


def _agg_mm(h, parts, wt, b, relu=False):
    grid = (N // BLOCK_M,)
    return pl.pallas_call(
        functools.partial(_agg_mm_body, relu=relu),
        grid=grid,
        in_specs=[
            pl.BlockSpec((BLOCK_M, D), lambda i: (i, 0)),
            pl.BlockSpec((BLOCK_M, D), lambda i: (i, 0)),
            pl.BlockSpec((BLOCK_M, D), lambda i: (i, 0)),
            pl.BlockSpec((D, D), lambda i: (0, 0)),
            pl.BlockSpec((1, D), lambda i: (0, 0)),
        ],
        out_specs=pl.BlockSpec((BLOCK_M, D), lambda i: (i, 0)),
        out_shape=jax.ShapeDtypeStruct((N, D), jnp.float32),
    )(h, parts[0], parts[1], wt, b.reshape(1, D))


def _sc_agg(h, src3, dst3):
    mesh = plsc.VectorSubcoreMesh(
        core_axis_name="c", subcore_axis_name="s", num_cores=NC, num_subcores=NS
    )

    @functools.partial(
        pl.kernel,
        out_type=jax.ShapeDtypeStruct((NC, N, D), jnp.float32),
        mesh=mesh,
        scratch_types=[
            pltpu.VMEM((E_TILE,), jnp.int32),
            pltpu.VMEM((N_CHUNKS, CHUNK), jnp.int32),
            pltpu.VMEM((CHUNK, D), jnp.float32),
            pltpu.VMEM((CHUNK, D), jnp.float32),
            pltpu.VMEM((ZROWS, D), jnp.float32),
            pltpu.VMEM_SHARED((N, D), jnp.float32),
            pltpu.SemaphoreType.DMA,
            pltpu.SemaphoreType.DMA,
        ],
    )
    def k(h_hbm, src_hbm, dst_hbm, out_hbm, src_v, dst_v, rows0, rows1,
          zeros_v, acc_sh, sem0, sem1):
        cid = lax.axis_index("c")
        sid = lax.axis_index("s")

        @pl.loop(0, ZROWS)
        def _(r):
            @pl.loop(0, D, step=16)
            def _(j):
                zeros_v[r, pl.ds(j, 16)] = jnp.zeros((16,), jnp.float32)

        row0 = sid * ROWS_PER_TILE

        @pl.loop(0, ROWS_PER_TILE, step=ZROWS)
        def _(r0):
            pltpu.sync_copy(zeros_v, acc_sh.at[pl.ds(row0 + r0, ZROWS)])

        @pl.when(sid == NS - 1)
        def _():
            pltpu.sync_copy(zeros_v, acc_sh.at[pl.ds(TAIL_ROW, TAIL)])

        plsc.subcore_barrier()

        wid = cid * NS + sid
        pltpu.sync_copy(src_hbm.at[pl.ds(wid * E_TILE, E_TILE)], src_v)
        pltpu.sync_copy(dst_hbm.at[wid], dst_v)

        def _src(j):
            return src_v.at[pl.ds(j * CHUNK, CHUNK)]

        pltpu.async_copy(h_hbm.at[_src(0)], rows0, sem0)

        @pl.loop(0, N_CHUNKS // 2)
        def _(i):
            j = 2 * i
            pltpu.make_async_copy(h_hbm.at[_src(j)], rows0, sem0).wait()
            pltpu.async_copy(h_hbm.at[_src(j + 1)], rows1, sem1)
            pltpu.sync_copy(rows0, acc_sh.at[dst_v.at[j]], add=True)

            pltpu.make_async_copy(h_hbm.at[_src(j + 1)], rows1, sem1).wait()
            pltpu.async_copy(h_hbm.at[_src(j + 2)], rows0, sem0)
            pltpu.sync_copy(rows1, acc_sh.at[dst_v.at[j + 1]], add=True)

        jt = N_CHUNKS - 1
        pltpu.make_async_copy(h_hbm.at[_src(jt)], rows0, sem0).wait()
        pltpu.sync_copy(rows0, acc_sh.at[dst_v.at[jt]], add=True)

        plsc.subcore_barrier()

        pltpu.sync_copy(
            acc_sh.at[pl.ds(row0, ROWS_PER_TILE)],
            out_hbm.at[cid, pl.ds(row0, ROWS_PER_TILE)],
        )

        @pl.when(sid == NS - 1)
        def _():
            pltpu.sync_copy(
                acc_sh.at[pl.ds(TAIL_ROW, TAIL)],
                out_hbm.at[cid, pl.ds(TAIL_ROW, TAIL)],
            )

    return k(h, src3, dst3)


@jax.jit
def kernel(x, edge_index, W_pre, b_pre, W1, b1, W2, b2):
    src3 = edge_index[0]
    dst3 = edge_index[1].reshape(NW, N_CHUNKS, CHUNK)
    h0 = _mm(x, W_pre.T, b_pre)
    p = _sc_agg(h0, src3, dst3)
    h1 = _agg_mm(h0, (p[0], p[1]), W1.T, b1, relu=True)
    q = _sc_agg(h1, src3, dst3)
    out = _agg_mm(h1, (q[0], q[1]), W2.T, b2)
    return out
